# baseline (speedup 1.0000x reference)
import functools

import jax
import jax.numpy as jnp
import numpy as np
from jax.experimental import pallas as pl
from jax.experimental.pallas import tpu as pltpu
from jax.experimental.pallas import tpu_sc as plsc

B = 1024
T = 200
H = 64
N = B * T
NR = 4
NC = 4
NO = 4
STATIC_OFFS = (0, 100000, 200000, 210000)
CAT_VOCAB = 1000

GATHER_W = 128
NB = 2048


def _sc_gather_all(stat_tab, stat_idx, cat_tab, cat_idx):
    mesh = plsc.VectorSubcoreMesh(core_axis_name="core", subcore_axis_name="subcore")

    @functools.partial(
        pl.kernel,
        out_type=(
            jax.ShapeDtypeStruct((4 * B, H), jnp.float32),
            jax.ShapeDtypeStruct((NC * N, H), jnp.float32),
        ),
        mesh=mesh,
        compiler_params=pltpu.CompilerParams(use_tc_tiling_on_sc=False),
    )
    def sc_kernel(stat_tab_hbm, stat_idx_hbm, cat_tab_hbm, cat_idx_hbm,
                  stat_out_hbm, cat_out_hbm):
        def stat_body(i_vmem, o_vmem):
            pltpu.sync_copy(stat_tab_hbm.at[i_vmem.at[0]], o_vmem)

        pltpu.emit_pipeline(
            stat_body,
            grid=(4 * B // GATHER_W,),
            in_specs=[pl.BlockSpec((1, GATHER_W), lambda i: (0, i))],
            out_specs=[pl.BlockSpec((GATHER_W, H), lambda i: (i, 0))],
            core_axis_name=("core", "subcore"),
            dimension_semantics=(pltpu.PARALLEL,),
        )(stat_idx_hbm, stat_out_hbm)

        def cat_body(i_vmem, o_vmem):
            pltpu.sync_copy(cat_tab_hbm.at[i_vmem.at[0]], o_vmem)

        pltpu.emit_pipeline(
            cat_body,
            grid=(NC * N // GATHER_W,),
            in_specs=[pl.BlockSpec((1, GATHER_W), lambda i: (0, i))],
            out_specs=[pl.BlockSpec((GATHER_W, H), lambda i: (i, 0))],
            core_axis_name=("core", "subcore"),
            dimension_semantics=(pltpu.PARALLEL,),
        )(cat_idx_hbm, cat_out_hbm)

    return sc_kernel(stat_tab, stat_idx, cat_tab, cat_idx)


def _tc_body(g_ref, xk_ref, xo_ref, r_ref, wk_ref, kb_ref, wo_ref, ob_ref,
             known_ref, obs_ref):
    g = g_ref[...]
    hi = g.astype(jnp.bfloat16)
    lo = (g - hi.astype(jnp.float32)).astype(jnp.bfloat16)
    r = r_ref[...]
    acc = jnp.dot(hi, r, preferred_element_type=jnp.float32)
    acc = acc + jnp.dot(lo, r, preferred_element_type=jnp.float32)
    xk = xk_ref[...]
    for c in range(NR):
        acc = acc + xk[:, c:c + 1] * wk_ref[c:c + 1, :]
    known_ref[...] = acc + kb_ref[...]
    xo = xo_ref[...]
    acc2 = xo[:, 0:1] * wo_ref[0:1, :]
    for c in range(1, NO):
        acc2 = acc2 + xo[:, c:c + 1] * wo_ref[c:c + 1, :]
    obs_ref[...] = acc2 + ob_ref[...]


def _interleave_rows(rows, width):
    return jnp.stack(rows, axis=-1).reshape(H * width)


def kernel(static_0, static_1, static_2, static_3, known_real_0, known_real_1, known_real_2, known_real_3, known_cat_0, known_cat_1, known_cat_2, known_cat_3, observed_0, observed_1, observed_2, observed_3, static_table_0, static_table_1, static_table_2, static_table_3, known_cat_table_0, known_cat_table_1, known_cat_table_2, known_cat_table_3, kr_W_0, kr_b_0, kr_W_1, kr_b_1, kr_W_2, kr_b_2, kr_W_3, kr_b_3, obs_W_0, obs_b_0, obs_W_1, obs_b_1, obs_W_2, obs_b_2, obs_W_3, obs_b_3):
    f32 = jnp.float32
    statics = [static_0, static_1, static_2, static_3]
    kcats = [known_cat_0, known_cat_1, known_cat_2, known_cat_3]
    kreals = [known_real_0, known_real_1, known_real_2, known_real_3]
    obs = [observed_0, observed_1, observed_2, observed_3]

    stat_tab = jnp.concatenate(
        [static_table_0, static_table_1, static_table_2, static_table_3], axis=0)
    stat_idx = jnp.stack(
        [s.astype(jnp.int32) + o for s, o in zip(statics, STATIC_OFFS)],
        axis=1).reshape(1, 4 * B)
    cat_tab = jnp.concatenate(
        [known_cat_table_0, known_cat_table_1, known_cat_table_2,
         known_cat_table_3], axis=0)
    cat_idx = jnp.stack(
        [k.astype(jnp.int32).reshape(N) + CAT_VOCAB * c
         for c, k in enumerate(kcats)], axis=1).reshape(1, NC * N)

    static_rows, cat_rows = _sc_gather_all(stat_tab, stat_idx, cat_tab, cat_idx)
    static_out = static_rows.reshape(B, 4, H)
    g_mat = cat_rows.reshape(N, NC * H)

    r_np = np.zeros((NC * H, 8 * H), np.float32)
    for c in range(NC):
        for h in range(H):
            r_np[c * H + h, h * 8 + NR + c] = 1.0
    r_mat = jnp.asarray(r_np, dtype=jnp.bfloat16)

    eye4_8 = jnp.asarray(np.pad(np.eye(4, dtype=np.float32), ((0, 0), (0, 4))))
    kr_w = jnp.stack([kr_W_0[0], kr_W_1[0], kr_W_2[0], kr_W_3[0]], axis=0)
    wk_int = (kr_w[:, :, None] * eye4_8[:, None, :]).reshape(NR, 8 * H)
    kb_int = _interleave_rows(
        [kr_b_0, kr_b_1, kr_b_2, kr_b_3] + [jnp.zeros((H,), f32)] * 4, 8
    ).reshape(1, 8 * H)

    eye4 = jnp.asarray(np.eye(4, dtype=np.float32))
    obs_w = jnp.stack([obs_W_0[0], obs_W_1[0], obs_W_2[0], obs_W_3[0]], axis=0)
    wo_int = (obs_w[:, :, None] * eye4[:, None, :]).reshape(NO, 4 * H)
    ob_int = _interleave_rows([obs_b_0, obs_b_1, obs_b_2, obs_b_3], 4
                              ).reshape(1, 4 * H)

    xk = jnp.stack([x.reshape(N) for x in kreals], axis=1)
    xo = jnp.stack([x.reshape(N) for x in obs], axis=1)

    grid = (N // NB,)
    known_flat, obs_flat = pl.pallas_call(
        _tc_body,
        grid=grid,
        in_specs=[
            pl.BlockSpec((NB, NC * H), lambda i: (i, 0)),
            pl.BlockSpec((NB, NR), lambda i: (i, 0)),
            pl.BlockSpec((NB, NO), lambda i: (i, 0)),
            pl.BlockSpec((NC * H, 8 * H), lambda i: (0, 0)),
            pl.BlockSpec((NR, 8 * H), lambda i: (0, 0)),
            pl.BlockSpec((1, 8 * H), lambda i: (0, 0)),
            pl.BlockSpec((NO, 4 * H), lambda i: (0, 0)),
            pl.BlockSpec((1, 4 * H), lambda i: (0, 0)),
        ],
        out_specs=[
            pl.BlockSpec((NB, 8 * H), lambda i: (i, 0)),
            pl.BlockSpec((NB, 4 * H), lambda i: (i, 0)),
        ],
        out_shape=[
            jax.ShapeDtypeStruct((N, 8 * H), f32),
            jax.ShapeDtypeStruct((N, 4 * H), f32),
        ],
    )(g_mat, xk, xo, r_mat, wk_int, kb_int, wo_int, ob_int)

    known = known_flat.reshape(B, T, H, 8)
    observed = obs_flat.reshape(B, T, H, 4)
    return (static_out, known, observed)


# speedup vs baseline: 2.7011x; 2.7011x over previous
import functools

import jax
import jax.numpy as jnp
import numpy as np
from jax.experimental import pallas as pl
from jax.experimental.pallas import tpu as pltpu
from jax.experimental.pallas import tpu_sc as plsc

B = 1024
T = 200
H = 64
N = B * T
NR = 4
NC = 4
NO = 4
STATIC_OFFS = (0, 100000, 200000, 210000)
CAT_VOCAB = 1000

GATHER_W = 128


def _sc_gather_all(stat_tab, stat_idx, cat_tab, cat_idx):
    mesh = plsc.VectorSubcoreMesh(core_axis_name="core", subcore_axis_name="subcore")

    @functools.partial(
        pl.kernel,
        out_type=(
            jax.ShapeDtypeStruct((4 * B, H), jnp.float32),
            jax.ShapeDtypeStruct((NC * N, H), jnp.float32),
        ),
        mesh=mesh,
        compiler_params=pltpu.CompilerParams(use_tc_tiling_on_sc=False),
    )
    def sc_kernel(stat_tab_hbm, stat_idx_hbm, cat_tab_hbm, cat_idx_hbm,
                  stat_out_hbm, cat_out_hbm):
        def stat_body(i_vmem, o_vmem):
            pltpu.sync_copy(stat_tab_hbm.at[i_vmem.at[0]], o_vmem)

        pltpu.emit_pipeline(
            stat_body,
            grid=(4 * B // GATHER_W,),
            in_specs=[pl.BlockSpec((1, GATHER_W), lambda i: (0, i))],
            out_specs=[pl.BlockSpec((GATHER_W, H), lambda i: (i, 0))],
            core_axis_name=("core", "subcore"),
            dimension_semantics=(pltpu.PARALLEL,),
        )(stat_idx_hbm, stat_out_hbm)

        def cat_body(i_vmem, o_vmem):
            pltpu.sync_copy(cat_tab_hbm.at[i_vmem.at[0]], o_vmem)

        pltpu.emit_pipeline(
            cat_body,
            grid=(NC * N // GATHER_W,),
            in_specs=[pl.BlockSpec((1, GATHER_W), lambda i: (0, i))],
            out_specs=[pl.BlockSpec((GATHER_W, H), lambda i: (i, 0))],
            core_axis_name=("core", "subcore"),
            dimension_semantics=(pltpu.PARALLEL,),
        )(cat_idx_hbm, cat_out_hbm)

    return sc_kernel(stat_tab, stat_idx, cat_tab, cat_idx)


def _tc_body(g_ref, xk_ref, xo_ref, p_ref, wk_ref, kb_ref, wo_ref, ob_ref,
             known_ref, obs_ref):
    g = g_ref[...]
    gt = jnp.transpose(g)
    hi = gt.astype(jnp.bfloat16)
    lo = (gt - hi.astype(jnp.float32)).astype(jnp.bfloat16)
    p = p_ref[...]
    acc = jnp.dot(p, hi, preferred_element_type=jnp.float32)
    acc = acc + jnp.dot(p, lo, preferred_element_type=jnp.float32)
    xk = xk_ref[0]
    for c in range(NR):
        acc = acc + wk_ref[:, c:c + 1] * xk[c:c + 1, :]
    known_ref[0] = acc + kb_ref[...]
    xo = xo_ref[0]
    acc2 = wo_ref[:, 0:1] * xo[0:1, :]
    for c in range(1, NO):
        acc2 = acc2 + wo_ref[:, c:c + 1] * xo[c:c + 1, :]
    obs_ref[0] = acc2 + ob_ref[...]


def _static_body(in_ref, out_ref):
    out_ref[0] = jnp.transpose(in_ref[0])


def kernel(static_0, static_1, static_2, static_3, known_real_0, known_real_1, known_real_2, known_real_3, known_cat_0, known_cat_1, known_cat_2, known_cat_3, observed_0, observed_1, observed_2, observed_3, static_table_0, static_table_1, static_table_2, static_table_3, known_cat_table_0, known_cat_table_1, known_cat_table_2, known_cat_table_3, kr_W_0, kr_b_0, kr_W_1, kr_b_1, kr_W_2, kr_b_2, kr_W_3, kr_b_3, obs_W_0, obs_b_0, obs_W_1, obs_b_1, obs_W_2, obs_b_2, obs_W_3, obs_b_3):
    f32 = jnp.float32
    statics = [static_0, static_1, static_2, static_3]
    kcats = [known_cat_0, known_cat_1, known_cat_2, known_cat_3]
    kreals = [known_real_0, known_real_1, known_real_2, known_real_3]
    obs = [observed_0, observed_1, observed_2, observed_3]

    stat_tab = jnp.concatenate(
        [static_table_0, static_table_1, static_table_2, static_table_3], axis=0)
    stat_idx = jnp.concatenate(
        [s.astype(jnp.int32) + o for s, o in zip(statics, STATIC_OFFS)]
    ).reshape(1, 4 * B)
    cat_tab = jnp.concatenate(
        [known_cat_table_0, known_cat_table_1, known_cat_table_2,
         known_cat_table_3], axis=0)
    cat_idx = jnp.stack(
        [k.astype(jnp.int32).T + CAT_VOCAB * c for c, k in enumerate(kcats)],
        axis=-1).reshape(1, NC * N)

    static_rows, cat_rows = _sc_gather_all(stat_tab, stat_idx, cat_tab, cat_idx)
    g_mat = cat_rows.reshape(N, NC * H)

    sout = pl.pallas_call(
        _static_body,
        grid=(4,),
        in_specs=[pl.BlockSpec((1, B, H), lambda j: (j, 0, 0))],
        out_specs=pl.BlockSpec((1, H, B), lambda j: (j, 0, 0)),
        out_shape=jax.ShapeDtypeStruct((4, H, B), f32),
    )(static_rows.reshape(4, B, H))
    static_out = sout.transpose(2, 0, 1)

    p_np = np.zeros((8 * H, NC * H), np.float32)
    for c in range(NC):
        for h in range(H):
            p_np[h * 8 + NR + c, c * H + h] = 1.0
    p_mat = jnp.asarray(p_np, dtype=jnp.bfloat16)

    eye84 = jnp.asarray(np.eye(8, 4, dtype=np.float32))
    kr_w = jnp.stack([kr_W_0[0], kr_W_1[0], kr_W_2[0], kr_W_3[0]], axis=1)
    wk_col = (kr_w[:, None, :] * eye84[None, :, :]).reshape(8 * H, NR)
    kb_col = jnp.stack(
        [kr_b_0, kr_b_1, kr_b_2, kr_b_3] + [jnp.zeros((H,), f32)] * 4,
        axis=-1).reshape(8 * H, 1)

    eye44 = jnp.asarray(np.eye(4, dtype=np.float32))
    obs_w = jnp.stack([obs_W_0[0], obs_W_1[0], obs_W_2[0], obs_W_3[0]], axis=1)
    wo_col = (obs_w[:, None, :] * eye44[None, :, :]).reshape(4 * H, NO)
    ob_col = jnp.stack([obs_b_0, obs_b_1, obs_b_2, obs_b_3],
                       axis=-1).reshape(4 * H, 1)

    xk = jnp.stack([x.T for x in kreals], axis=1)
    xo = jnp.stack([x.T for x in obs], axis=1)

    known_p, obs_p = pl.pallas_call(
        _tc_body,
        grid=(T,),
        in_specs=[
            pl.BlockSpec((B, NC * H), lambda t: (t, 0)),
            pl.BlockSpec((1, NR, B), lambda t: (t, 0, 0)),
            pl.BlockSpec((1, NO, B), lambda t: (t, 0, 0)),
            pl.BlockSpec((8 * H, NC * H), lambda t: (0, 0)),
            pl.BlockSpec((8 * H, NR), lambda t: (0, 0)),
            pl.BlockSpec((8 * H, 1), lambda t: (0, 0)),
            pl.BlockSpec((4 * H, NO), lambda t: (0, 0)),
            pl.BlockSpec((4 * H, 1), lambda t: (0, 0)),
        ],
        out_specs=[
            pl.BlockSpec((1, 8 * H, B), lambda t: (t, 0, 0)),
            pl.BlockSpec((1, 4 * H, B), lambda t: (t, 0, 0)),
        ],
        out_shape=[
            jax.ShapeDtypeStruct((T, 8 * H, B), f32),
            jax.ShapeDtypeStruct((T, 4 * H, B), f32),
        ],
    )(g_mat, xk, xo, p_mat, wk_col, kb_col, wo_col, ob_col)

    known = known_p.reshape(T, H, 8, B).transpose(3, 0, 1, 2)
    observed = obs_p.reshape(T, H, 4, B).transpose(3, 0, 1, 2)
    return (static_out, known, observed)


# speedup vs baseline: 2.8203x; 1.0441x over previous
import functools

import jax
import jax.numpy as jnp
import numpy as np
from jax.experimental import pallas as pl
from jax.experimental.pallas import tpu as pltpu
from jax.experimental.pallas import tpu_sc as plsc

B = 1024
T = 200
H = 64
N = B * T
NR = 4
NC = 4
NO = 4
CAT_VOCAB = 1000

GATHER_W = 128


def _sc_gather_all(st0, st1, st2, st3, stat_idx, cat_tab, cat_idx):
    mesh = plsc.VectorSubcoreMesh(core_axis_name="core", subcore_axis_name="subcore")

    @functools.partial(
        pl.kernel,
        out_type=(
            jax.ShapeDtypeStruct((4 * B, H), jnp.float32),
            jax.ShapeDtypeStruct((NC * N, H), jnp.float32),
        ),
        mesh=mesh,
        compiler_params=pltpu.CompilerParams(use_tc_tiling_on_sc=False),
    )
    def sc_kernel(t0_hbm, t1_hbm, t2_hbm, t3_hbm, stat_idx_hbm,
                  cat_tab_hbm, cat_idx_hbm, stat_out_hbm, cat_out_hbm):
        for j, tab in enumerate((t0_hbm, t1_hbm, t2_hbm, t3_hbm)):
            def stat_body(i_vmem, o_vmem, tab=tab):
                pltpu.sync_copy(tab.at[i_vmem.at[0]], o_vmem)

            pltpu.emit_pipeline(
                stat_body,
                grid=(B // GATHER_W,),
                in_specs=[pl.BlockSpec((1, GATHER_W), lambda i, j=j: (j, i))],
                out_specs=[pl.BlockSpec((GATHER_W, H),
                                        lambda i, j=j: (j * (B // GATHER_W) + i, 0))],
                core_axis_name=("core", "subcore"),
                dimension_semantics=(pltpu.PARALLEL,),
            )(stat_idx_hbm, stat_out_hbm)

        def cat_body(i_vmem, o_vmem):
            pltpu.sync_copy(cat_tab_hbm.at[i_vmem.at[0]], o_vmem)

        pltpu.emit_pipeline(
            cat_body,
            grid=(NC * N // GATHER_W,),
            in_specs=[pl.BlockSpec((1, GATHER_W), lambda i: (0, i))],
            out_specs=[pl.BlockSpec((GATHER_W, H), lambda i: (i, 0))],
            core_axis_name=("core", "subcore"),
            dimension_semantics=(pltpu.PARALLEL,),
        )(cat_idx_hbm, cat_out_hbm)

    return sc_kernel(st0, st1, st2, st3, stat_idx, cat_tab, cat_idx)


def _even_odd(x):
    e = jnp.concatenate([x[:, 0:128], x[:, 256:384], x[:, 512:640],
                         x[:, 768:896]], axis=1)
    o = jnp.concatenate([x[:, 128:256], x[:, 384:512], x[:, 640:768],
                         x[:, 896:1024]], axis=1)
    return e, o


def _tc_body(g_ref, xk_ref, xo_ref, p_ref, wk_ref, kb_ref, wo_ref, ob_ref,
             known_ref, obs_ref):
    g4 = g_ref[...].reshape(NC, B, H)
    gt = jnp.transpose(g4, (0, 2, 1)).reshape(NC * H, B)
    hi = gt.astype(jnp.bfloat16)
    lo = (gt - hi.astype(jnp.float32)).astype(jnp.bfloat16)
    p = p_ref[...]
    acc = jnp.dot(p, hi, preferred_element_type=jnp.float32)
    acc = acc + jnp.dot(p, lo, preferred_element_type=jnp.float32)
    xk = xk_ref[0]
    for c in range(NR):
        acc = acc + wk_ref[:, c:c + 1] * xk[c:c + 1, :]
    known_ref[0] = acc + kb_ref[...]
    xo = xo_ref[0]
    acc2 = wo_ref[:, 0:1] * xo[0:1, :]
    for c in range(1, NO):
        acc2 = acc2 + wo_ref[:, c:c + 1] * xo[c:c + 1, :]
    obs_ref[0] = acc2 + ob_ref[...]


def _static_body(in_ref, out_ref):
    out_ref[0] = jnp.transpose(in_ref[0])


def kernel(static_0, static_1, static_2, static_3, known_real_0, known_real_1, known_real_2, known_real_3, known_cat_0, known_cat_1, known_cat_2, known_cat_3, observed_0, observed_1, observed_2, observed_3, static_table_0, static_table_1, static_table_2, static_table_3, known_cat_table_0, known_cat_table_1, known_cat_table_2, known_cat_table_3, kr_W_0, kr_b_0, kr_W_1, kr_b_1, kr_W_2, kr_b_2, kr_W_3, kr_b_3, obs_W_0, obs_b_0, obs_W_1, obs_b_1, obs_W_2, obs_b_2, obs_W_3, obs_b_3):
    f32 = jnp.float32
    statics = [static_0, static_1, static_2, static_3]
    kcats = [known_cat_0, known_cat_1, known_cat_2, known_cat_3]
    kreals = [known_real_0, known_real_1, known_real_2, known_real_3]
    obs = [observed_0, observed_1, observed_2, observed_3]

    stat_idx = jnp.stack([s.astype(jnp.int32) for s in statics], axis=0)
    cat_tab = jnp.concatenate(
        [known_cat_table_0, known_cat_table_1, known_cat_table_2,
         known_cat_table_3], axis=0)
    cat_idx = jnp.concatenate(
        [k.astype(jnp.int32).T.reshape(N) + CAT_VOCAB * c
         for c, k in enumerate(kcats)]).reshape(1, NC * N)

    static_rows, cat_rows = _sc_gather_all(
        static_table_0, static_table_1, static_table_2, static_table_3,
        stat_idx, cat_tab, cat_idx)
    g_mat = cat_rows.reshape(NC, T, B, H)

    sout = pl.pallas_call(
        _static_body,
        grid=(4,),
        in_specs=[pl.BlockSpec((1, B, H), lambda j: (j, 0, 0))],
        out_specs=pl.BlockSpec((1, H, B), lambda j: (j, 0, 0)),
        out_shape=jax.ShapeDtypeStruct((4, H, B), f32),
    )(static_rows.reshape(4, B, H))
    static_out = sout.transpose(2, 0, 1)

    p_np = np.zeros((8 * H, NC * H), np.float32)
    for c in range(NC):
        for h in range(H):
            p_np[h * 8 + NR + c, c * H + h] = 1.0
    p_mat = jnp.asarray(p_np, dtype=jnp.bfloat16)

    eye84 = jnp.asarray(np.eye(8, 4, dtype=np.float32))
    kr_w = jnp.stack([kr_W_0[0], kr_W_1[0], kr_W_2[0], kr_W_3[0]], axis=1)
    wk_col = (kr_w[:, None, :] * eye84[None, :, :]).reshape(8 * H, NR)
    kb_col = jnp.stack(
        [kr_b_0, kr_b_1, kr_b_2, kr_b_3] + [jnp.zeros((H,), f32)] * 4,
        axis=-1).reshape(8 * H, 1)

    eye44 = jnp.asarray(np.eye(4, dtype=np.float32))
    obs_w = jnp.stack([obs_W_0[0], obs_W_1[0], obs_W_2[0], obs_W_3[0]], axis=1)
    wo_col = (obs_w[:, None, :] * eye44[None, :, :]).reshape(4 * H, NO)
    ob_col = jnp.stack([obs_b_0, obs_b_1, obs_b_2, obs_b_3],
                       axis=-1).reshape(4 * H, 1)

    xk = jnp.stack([x.T for x in kreals], axis=1)
    xo = jnp.stack([x.T for x in obs], axis=1)

    known_p, obs_p = pl.pallas_call(
        _tc_body,
        grid=(T,),
        in_specs=[
            pl.BlockSpec((NC, 1, B, H), lambda t: (0, t, 0, 0)),
            pl.BlockSpec((1, NR, B), lambda t: (t, 0, 0)),
            pl.BlockSpec((1, NO, B), lambda t: (t, 0, 0)),
            pl.BlockSpec((8 * H, NC * H), lambda t: (0, 0)),
            pl.BlockSpec((8 * H, NR), lambda t: (0, 0)),
            pl.BlockSpec((8 * H, 1), lambda t: (0, 0)),
            pl.BlockSpec((4 * H, NO), lambda t: (0, 0)),
            pl.BlockSpec((4 * H, 1), lambda t: (0, 0)),
        ],
        out_specs=[
            pl.BlockSpec((1, 8 * H, B), lambda t: (t, 0, 0)),
            pl.BlockSpec((1, 4 * H, B), lambda t: (t, 0, 0)),
        ],
        out_shape=[
            jax.ShapeDtypeStruct((T, 8 * H, B), f32),
            jax.ShapeDtypeStruct((T, 4 * H, B), f32),
        ],
    )(g_mat, xk, xo, p_mat, wk_col, kb_col, wo_col, ob_col)

    known = known_p.reshape(T, H, 8, B).transpose(3, 0, 1, 2)
    observed = obs_p.reshape(T, H, 4, B).transpose(3, 0, 1, 2)
    return (static_out, known, observed)


# speedup vs baseline: 2.9821x; 1.0574x over previous
import functools

import jax
import jax.numpy as jnp
import numpy as np
from jax.experimental import pallas as pl
from jax.experimental.pallas import tpu as pltpu
from jax.experimental.pallas import tpu_sc as plsc

B = 1024
T = 200
H = 64
N = B * T
NR = 4
NC = 4
NO = 4
CAT_VOCAB = 1000

GATHER_W = 128


def _sc_gather_all(st0, st1, st2, st3, stat_idx, cat_tab, cat_idx):
    mesh = plsc.VectorSubcoreMesh(core_axis_name="core", subcore_axis_name="subcore")

    @functools.partial(
        pl.kernel,
        out_type=(
            jax.ShapeDtypeStruct((4 * B, H), jnp.float32),
            jax.ShapeDtypeStruct((NC * N, H), jnp.float32),
        ),
        mesh=mesh,
        compiler_params=pltpu.CompilerParams(use_tc_tiling_on_sc=False),
    )
    def sc_kernel(t0_hbm, t1_hbm, t2_hbm, t3_hbm, stat_idx_hbm,
                  cat_tab_hbm, cat_idx_hbm, stat_out_hbm, cat_out_hbm):
        for j, tab in enumerate((t0_hbm, t1_hbm, t2_hbm, t3_hbm)):
            def stat_body(i_vmem, o_vmem, tab=tab):
                pltpu.sync_copy(tab.at[i_vmem.at[0]], o_vmem)

            pltpu.emit_pipeline(
                stat_body,
                grid=(B // GATHER_W,),
                in_specs=[pl.BlockSpec((1, GATHER_W), lambda i, j=j: (j, i))],
                out_specs=[pl.BlockSpec((GATHER_W, H),
                                        lambda i, j=j: (j * (B // GATHER_W) + i, 0))],
                core_axis_name=("core", "subcore"),
                dimension_semantics=(pltpu.PARALLEL,),
            )(stat_idx_hbm, stat_out_hbm)

        def cat_body(i_vmem, o_vmem):
            pltpu.sync_copy(cat_tab_hbm.at[i_vmem.at[0]], o_vmem)

        pltpu.emit_pipeline(
            cat_body,
            grid=(NC * N // GATHER_W,),
            in_specs=[pl.BlockSpec((1, GATHER_W), lambda i: (0, i))],
            out_specs=[pl.BlockSpec((GATHER_W, H), lambda i: (i, 0))],
            core_axis_name=("core", "subcore"),
            dimension_semantics=(pltpu.PARALLEL,),
        )(cat_idx_hbm, cat_out_hbm)

    return sc_kernel(st0, st1, st2, st3, stat_idx, cat_tab, cat_idx)


def _tc_body(g_ref, xk0, xk1, xk2, xk3, xo0, xo1, xo2, xo3, p_ref, wk_ref,
             kb_ref, wo_ref, ob_ref, known_ref, obs_ref):
    raw = g_ref[...].reshape(NC, B // 2, 2 * H)
    ga = raw[:, :, 0:H]
    gb = raw[:, :, H:2 * H]
    ta = jnp.transpose(ga, (0, 2, 1))
    tb = jnp.transpose(gb, (0, 2, 1))
    gt = jnp.concatenate([ta, tb], axis=-1).reshape(NC * H, B)
    hi = gt.astype(jnp.bfloat16)
    lo = (gt - hi.astype(jnp.float32)).astype(jnp.bfloat16)
    p = p_ref[...]
    acc = jnp.dot(p, hi, preferred_element_type=jnp.float32)
    acc = acc + jnp.dot(p, lo, preferred_element_type=jnp.float32)
    for c, xk in enumerate((xk0, xk1, xk2, xk3)):
        acc = acc + wk_ref[:, c:c + 1] * xk[0]
    known_ref[0] = acc + kb_ref[...]
    acc2 = wo_ref[:, 0:1] * xo0[0]
    for c, xo in enumerate((xo1, xo2, xo3)):
        acc2 = acc2 + wo_ref[:, c + 1:c + 2] * xo[0]
    obs_ref[0] = acc2 + ob_ref[...]


def _static_body(in_ref, out_ref):
    out_ref[0] = jnp.transpose(in_ref[0])


def kernel(static_0, static_1, static_2, static_3, known_real_0, known_real_1, known_real_2, known_real_3, known_cat_0, known_cat_1, known_cat_2, known_cat_3, observed_0, observed_1, observed_2, observed_3, static_table_0, static_table_1, static_table_2, static_table_3, known_cat_table_0, known_cat_table_1, known_cat_table_2, known_cat_table_3, kr_W_0, kr_b_0, kr_W_1, kr_b_1, kr_W_2, kr_b_2, kr_W_3, kr_b_3, obs_W_0, obs_b_0, obs_W_1, obs_b_1, obs_W_2, obs_b_2, obs_W_3, obs_b_3):
    f32 = jnp.float32
    statics = [static_0, static_1, static_2, static_3]
    kcats = [known_cat_0, known_cat_1, known_cat_2, known_cat_3]
    kreals = [known_real_0, known_real_1, known_real_2, known_real_3]
    obs = [observed_0, observed_1, observed_2, observed_3]

    stat_idx = jnp.stack([s.astype(jnp.int32) for s in statics], axis=0)
    cat_tab = jnp.concatenate(
        [known_cat_table_0, known_cat_table_1, known_cat_table_2,
         known_cat_table_3], axis=0)
    cat_idx = jnp.concatenate(
        [jnp.stack([k.astype(jnp.int32).T[:, :B // 2],
                    k.astype(jnp.int32).T[:, B // 2:]], axis=-1
                   ).reshape(N) + CAT_VOCAB * c
         for c, k in enumerate(kcats)]).reshape(1, NC * N)

    static_rows, cat_rows = _sc_gather_all(
        static_table_0, static_table_1, static_table_2, static_table_3,
        stat_idx, cat_tab, cat_idx)
    g_mat = cat_rows.reshape(NC, T, (B // 2) * (2 * H) // 128, 128)

    sout = pl.pallas_call(
        _static_body,
        grid=(4,),
        in_specs=[pl.BlockSpec((1, B, H), lambda j: (j, 0, 0))],
        out_specs=pl.BlockSpec((1, H, B), lambda j: (j, 0, 0)),
        out_shape=jax.ShapeDtypeStruct((4, H, B), f32),
    )(static_rows.reshape(4, B, H))
    static_out = sout.transpose(2, 0, 1)

    p_np = np.zeros((8 * H, NC * H), np.float32)
    for c in range(NC):
        for h in range(H):
            p_np[h * 8 + NR + c, c * H + h] = 1.0
    p_mat = jnp.asarray(p_np, dtype=jnp.bfloat16)

    eye84 = jnp.asarray(np.eye(8, 4, dtype=np.float32))
    kr_w = jnp.stack([kr_W_0[0], kr_W_1[0], kr_W_2[0], kr_W_3[0]], axis=1)
    wk_col = (kr_w[:, None, :] * eye84[None, :, :]).reshape(8 * H, NR)
    kb_col = jnp.stack(
        [kr_b_0, kr_b_1, kr_b_2, kr_b_3] + [jnp.zeros((H,), f32)] * 4,
        axis=-1).reshape(8 * H, 1)

    eye44 = jnp.asarray(np.eye(4, dtype=np.float32))
    obs_w = jnp.stack([obs_W_0[0], obs_W_1[0], obs_W_2[0], obs_W_3[0]], axis=1)
    wo_col = (obs_w[:, None, :] * eye44[None, :, :]).reshape(4 * H, NO)
    ob_col = jnp.stack([obs_b_0, obs_b_1, obs_b_2, obs_b_3],
                       axis=-1).reshape(4 * H, 1)

    xks = [x.T.reshape(T, 1, B) for x in kreals]
    xos = [x.T.reshape(T, 1, B) for x in obs]

    xspec = pl.BlockSpec((1, 1, B), lambda t: (t, 0, 0))
    known_p, obs_p = pl.pallas_call(
        _tc_body,
        grid=(T,),
        in_specs=[
            pl.BlockSpec((NC, 1, (B // 2) * (2 * H) // 128, 128),
                         lambda t: (0, t, 0, 0)),
            xspec, xspec, xspec, xspec, xspec, xspec, xspec, xspec,
            pl.BlockSpec((8 * H, NC * H), lambda t: (0, 0)),
            pl.BlockSpec((8 * H, NR), lambda t: (0, 0)),
            pl.BlockSpec((8 * H, 1), lambda t: (0, 0)),
            pl.BlockSpec((4 * H, NO), lambda t: (0, 0)),
            pl.BlockSpec((4 * H, 1), lambda t: (0, 0)),
        ],
        out_specs=[
            pl.BlockSpec((1, 8 * H, B), lambda t: (t, 0, 0)),
            pl.BlockSpec((1, 4 * H, B), lambda t: (t, 0, 0)),
        ],
        out_shape=[
            jax.ShapeDtypeStruct((T, 8 * H, B), f32),
            jax.ShapeDtypeStruct((T, 4 * H, B), f32),
        ],
    )(g_mat, *xks, *xos, p_mat, wk_col, kb_col, wo_col, ob_col)

    known = known_p.reshape(T, H, 8, B).transpose(3, 0, 1, 2)
    observed = obs_p.reshape(T, H, 4, B).transpose(3, 0, 1, 2)
    return (static_out, known, observed)


# speedup vs baseline: 2.9824x; 1.0001x over previous
import functools

import jax
import jax.numpy as jnp
import numpy as np
from jax.experimental import pallas as pl
from jax.experimental.pallas import tpu as pltpu
from jax.experimental.pallas import tpu_sc as plsc

B = 1024
T = 200
H = 64
N = B * T
NR = 4
NC = 4
NO = 4
CAT_VOCAB = 1000

GATHER_W = 128


def _sc_gather_all(st0, st1, st2, st3, stat_idx, cat_tab, cat_idx):
    mesh = plsc.VectorSubcoreMesh(core_axis_name="core", subcore_axis_name="subcore")

    @functools.partial(
        pl.kernel,
        out_type=(
            jax.ShapeDtypeStruct((4 * B, H), jnp.float32),
            jax.ShapeDtypeStruct((NC * N, H), jnp.float32),
        ),
        mesh=mesh,
        compiler_params=pltpu.CompilerParams(use_tc_tiling_on_sc=False),
    )
    def sc_kernel(t0_hbm, t1_hbm, t2_hbm, t3_hbm, stat_idx_hbm,
                  cat_tab_hbm, cat_idx_hbm, stat_out_hbm, cat_out_hbm):
        for j, tab in enumerate((t0_hbm, t1_hbm, t2_hbm, t3_hbm)):
            def stat_body(i_vmem, o_vmem, tab=tab):
                pltpu.sync_copy(tab.at[i_vmem.at[0]], o_vmem)

            pltpu.emit_pipeline(
                stat_body,
                grid=(B // GATHER_W,),
                in_specs=[pl.BlockSpec((1, GATHER_W), lambda i, j=j: (j, i))],
                out_specs=[pl.BlockSpec((GATHER_W, H),
                                        lambda i, j=j: (j * (B // GATHER_W) + i, 0))],
                core_axis_name=("core", "subcore"),
                dimension_semantics=(pltpu.PARALLEL,),
            )(stat_idx_hbm, stat_out_hbm)

        def cat_body(i_vmem, o_vmem):
            pltpu.sync_copy(cat_tab_hbm.at[i_vmem.at[0]], o_vmem)

        pltpu.emit_pipeline(
            cat_body,
            grid=(NC * N // GATHER_W,),
            in_specs=[pl.BlockSpec((1, GATHER_W), lambda i: (0, i))],
            out_specs=[pl.BlockSpec((GATHER_W, H), lambda i: (i, 0))],
            core_axis_name=("core", "subcore"),
            dimension_semantics=(pltpu.PARALLEL,),
        )(cat_idx_hbm, cat_out_hbm)

    return sc_kernel(st0, st1, st2, st3, stat_idx, cat_tab, cat_idx)


def _tc_body(g_ref, xk0, xk1, xk2, xk3, xo0, xo1, xo2, xo3, p_ref, wk_ref,
             kb_ref, wo_ref, ob_ref, known_ref, obs_ref):
    raw = g_ref[...].reshape(NC, B // 2, 2 * H)
    ga = raw[:, :, 0:H]
    gb = raw[:, :, H:2 * H]
    ta = jnp.transpose(ga, (0, 2, 1))
    tb = jnp.transpose(gb, (0, 2, 1))
    gt = jnp.concatenate([ta, tb], axis=-1).reshape(NC * H, B)
    hi = gt.astype(jnp.bfloat16)
    lo = (gt - hi.astype(jnp.float32)).astype(jnp.bfloat16)
    p = p_ref[...]
    acc = jnp.dot(p, hi, preferred_element_type=jnp.float32)
    acc = acc + jnp.dot(p, lo, preferred_element_type=jnp.float32)
    for c, xk in enumerate((xk0, xk1, xk2, xk3)):
        acc = acc + wk_ref[:, c:c + 1] * xk[0]
    known_ref[0] = acc + kb_ref[...]
    acc2 = wo_ref[:, 0:1] * xo0[0]
    for c, xo in enumerate((xo1, xo2, xo3)):
        acc2 = acc2 + wo_ref[:, c + 1:c + 2] * xo[0]
    obs_ref[0] = acc2 + ob_ref[...]


def _static_body(in_ref, out_ref):
    out_ref[0] = jnp.transpose(in_ref[0])


def kernel(static_0, static_1, static_2, static_3, known_real_0, known_real_1, known_real_2, known_real_3, known_cat_0, known_cat_1, known_cat_2, known_cat_3, observed_0, observed_1, observed_2, observed_3, static_table_0, static_table_1, static_table_2, static_table_3, known_cat_table_0, known_cat_table_1, known_cat_table_2, known_cat_table_3, kr_W_0, kr_b_0, kr_W_1, kr_b_1, kr_W_2, kr_b_2, kr_W_3, kr_b_3, obs_W_0, obs_b_0, obs_W_1, obs_b_1, obs_W_2, obs_b_2, obs_W_3, obs_b_3):
    f32 = jnp.float32
    statics = [static_0, static_1, static_2, static_3]
    kcats = [known_cat_0, known_cat_1, known_cat_2, known_cat_3]
    kreals = [known_real_0, known_real_1, known_real_2, known_real_3]
    obs = [observed_0, observed_1, observed_2, observed_3]

    stat_idx = jnp.stack([s.astype(jnp.int32) for s in statics], axis=0)
    cat_tab = jnp.concatenate(
        [known_cat_table_0, known_cat_table_1, known_cat_table_2,
         known_cat_table_3], axis=0)
    cat_idx = jnp.concatenate(
        [(k.astype(jnp.int32).T.reshape(T, 2, B // 2).transpose(0, 2, 1)
          .reshape(N)) + CAT_VOCAB * c
         for c, k in enumerate(kcats)]).reshape(1, NC * N)

    static_rows, cat_rows = _sc_gather_all(
        static_table_0, static_table_1, static_table_2, static_table_3,
        stat_idx, cat_tab, cat_idx)
    g_mat = cat_rows.reshape(NC, T, (B // 2) * (2 * H) // 128, 128)

    sout = pl.pallas_call(
        _static_body,
        grid=(4,),
        in_specs=[pl.BlockSpec((1, B, H), lambda j: (j, 0, 0))],
        out_specs=pl.BlockSpec((1, H, B), lambda j: (j, 0, 0)),
        out_shape=jax.ShapeDtypeStruct((4, H, B), f32),
    )(static_rows.reshape(4, B, H))
    static_out = sout.transpose(2, 0, 1)

    p_np = np.zeros((8 * H, NC * H), np.float32)
    for c in range(NC):
        for h in range(H):
            p_np[h * 8 + NR + c, c * H + h] = 1.0
    p_mat = jnp.asarray(p_np, dtype=jnp.bfloat16)

    eye84 = jnp.asarray(np.eye(8, 4, dtype=np.float32))
    kr_w = jnp.stack([kr_W_0[0], kr_W_1[0], kr_W_2[0], kr_W_3[0]], axis=1)
    wk_col = (kr_w[:, None, :] * eye84[None, :, :]).reshape(8 * H, NR)
    kb_col = jnp.stack(
        [kr_b_0, kr_b_1, kr_b_2, kr_b_3] + [jnp.zeros((H,), f32)] * 4,
        axis=-1).reshape(8 * H, 1)

    eye44 = jnp.asarray(np.eye(4, dtype=np.float32))
    obs_w = jnp.stack([obs_W_0[0], obs_W_1[0], obs_W_2[0], obs_W_3[0]], axis=1)
    wo_col = (obs_w[:, None, :] * eye44[None, :, :]).reshape(4 * H, NO)
    ob_col = jnp.stack([obs_b_0, obs_b_1, obs_b_2, obs_b_3],
                       axis=-1).reshape(4 * H, 1)

    xks = [x.T.reshape(T, 1, B) for x in kreals]
    xos = [x.T.reshape(T, 1, B) for x in obs]

    xspec = pl.BlockSpec((1, 1, B), lambda t: (t, 0, 0))
    known_p, obs_p = pl.pallas_call(
        _tc_body,
        grid=(T,),
        in_specs=[
            pl.BlockSpec((NC, 1, (B // 2) * (2 * H) // 128, 128),
                         lambda t: (0, t, 0, 0)),
            xspec, xspec, xspec, xspec, xspec, xspec, xspec, xspec,
            pl.BlockSpec((8 * H, NC * H), lambda t: (0, 0)),
            pl.BlockSpec((8 * H, NR), lambda t: (0, 0)),
            pl.BlockSpec((8 * H, 1), lambda t: (0, 0)),
            pl.BlockSpec((4 * H, NO), lambda t: (0, 0)),
            pl.BlockSpec((4 * H, 1), lambda t: (0, 0)),
        ],
        out_specs=[
            pl.BlockSpec((1, 8 * H, B), lambda t: (t, 0, 0)),
            pl.BlockSpec((1, 4 * H, B), lambda t: (t, 0, 0)),
        ],
        out_shape=[
            jax.ShapeDtypeStruct((T, 8 * H, B), f32),
            jax.ShapeDtypeStruct((T, 4 * H, B), f32),
        ],
    )(g_mat, *xks, *xos, p_mat, wk_col, kb_col, wo_col, ob_col)

    known = known_p.reshape(T, H, 8, B).transpose(3, 0, 1, 2)
    observed = obs_p.reshape(T, H, 4, B).transpose(3, 0, 1, 2)
    return (static_out, known, observed)


# speedup vs baseline: 3.6265x; 1.2160x over previous
import functools

import jax
import jax.numpy as jnp
import numpy as np
from jax.experimental import pallas as pl
from jax.experimental.pallas import tpu as pltpu
from jax.experimental.pallas import tpu_sc as plsc

B = 1024
T = 200
H = 64
N = B * T
NR = 4
NC = 4
NO = 4
CAT_VOCAB = 1000

GATHER_W = 128


def _sc_gather_all(st0, st1, st2, st3, stat_idx, cat_tab, cat_idx):
    mesh = plsc.VectorSubcoreMesh(core_axis_name="core", subcore_axis_name="subcore")

    @functools.partial(
        pl.kernel,
        out_type=(
            jax.ShapeDtypeStruct((4 * B, H), jnp.float32),
            jax.ShapeDtypeStruct((NC * T * (B // 2), 2, H), jnp.float32),
        ),
        mesh=mesh,
        compiler_params=pltpu.CompilerParams(use_tc_tiling_on_sc=False),
    )
    def sc_kernel(t0_hbm, t1_hbm, t2_hbm, t3_hbm, stat_idx_hbm,
                  cat_tab_hbm, cat_idx_hbm, stat_out_hbm, cat_out_hbm):
        for j, tab in enumerate((t0_hbm, t1_hbm, t2_hbm, t3_hbm)):
            def stat_body(i_vmem, o_vmem, tab=tab):
                pltpu.sync_copy(tab.at[i_vmem.at[0]], o_vmem)

            pltpu.emit_pipeline(
                stat_body,
                grid=(B // GATHER_W,),
                in_specs=[pl.BlockSpec((1, GATHER_W), lambda i, j=j: (j, i))],
                out_specs=[pl.BlockSpec((GATHER_W, H),
                                        lambda i, j=j: (j * (B // GATHER_W) + i, 0))],
                core_axis_name=("core", "subcore"),
                dimension_semantics=(pltpu.PARALLEL,),
            )(stat_idx_hbm, stat_out_hbm)

        def cat_body(i_vmem, o_vmem):
            pltpu.sync_copy(cat_tab_hbm.at[i_vmem.at[0]], o_vmem.at[:, 0])

        pltpu.emit_pipeline(
            cat_body,
            grid=(NC * N // GATHER_W,),
            in_specs=[pl.BlockSpec((1, GATHER_W), lambda w: (0, w))],
            out_specs=[pl.BlockSpec(
                (GATHER_W, 1, H),
                lambda w: (w // 8 * (B // 2 // GATHER_W) + w % 4, w % 8 // 4, 0))],
            core_axis_name=("core", "subcore"),
            dimension_semantics=(pltpu.PARALLEL,),
        )(cat_idx_hbm, cat_out_hbm)

    return sc_kernel(st0, st1, st2, st3, stat_idx, cat_tab, cat_idx)


def _tc_body(g_ref, xk0, xk1, xk2, xk3, xo0, xo1, xo2, xo3, p_ref, wk_ref,
             kb_ref, wo_ref, ob_ref, known_ref, obs_ref):
    raw = g_ref[...].reshape(NC, B // 2, 2 * H)
    ga = raw[:, :, 0:H]
    gb = raw[:, :, H:2 * H]
    ta = jnp.transpose(ga, (0, 2, 1))
    tb = jnp.transpose(gb, (0, 2, 1))
    gt = jnp.concatenate([ta, tb], axis=-1).reshape(NC * H, B)
    hi = gt.astype(jnp.bfloat16)
    lo = (gt - hi.astype(jnp.float32)).astype(jnp.bfloat16)
    p = p_ref[...]
    acc = jnp.dot(p, hi, preferred_element_type=jnp.float32)
    acc = acc + jnp.dot(p, lo, preferred_element_type=jnp.float32)
    for c, xk in enumerate((xk0, xk1, xk2, xk3)):
        acc = acc + wk_ref[:, c:c + 1] * xk[0]
    known_ref[0] = acc + kb_ref[...]
    acc2 = wo_ref[:, 0:1] * xo0[0]
    for c, xo in enumerate((xo1, xo2, xo3)):
        acc2 = acc2 + wo_ref[:, c + 1:c + 2] * xo[0]
    obs_ref[0] = acc2 + ob_ref[...]


def _static_body(in_ref, out_ref):
    out_ref[0] = jnp.transpose(in_ref[0])


def kernel(static_0, static_1, static_2, static_3, known_real_0, known_real_1, known_real_2, known_real_3, known_cat_0, known_cat_1, known_cat_2, known_cat_3, observed_0, observed_1, observed_2, observed_3, static_table_0, static_table_1, static_table_2, static_table_3, known_cat_table_0, known_cat_table_1, known_cat_table_2, known_cat_table_3, kr_W_0, kr_b_0, kr_W_1, kr_b_1, kr_W_2, kr_b_2, kr_W_3, kr_b_3, obs_W_0, obs_b_0, obs_W_1, obs_b_1, obs_W_2, obs_b_2, obs_W_3, obs_b_3):
    f32 = jnp.float32
    statics = [static_0, static_1, static_2, static_3]
    kcats = [known_cat_0, known_cat_1, known_cat_2, known_cat_3]
    kreals = [known_real_0, known_real_1, known_real_2, known_real_3]
    obs = [observed_0, observed_1, observed_2, observed_3]

    stat_idx = jnp.stack([s.astype(jnp.int32) for s in statics], axis=0)
    cat_tab = jnp.concatenate(
        [known_cat_table_0, known_cat_table_1, known_cat_table_2,
         known_cat_table_3], axis=0)
    cat_idx = jnp.concatenate(
        [k.astype(jnp.int32).T.reshape(N) + CAT_VOCAB * c
         for c, k in enumerate(kcats)]).reshape(1, NC * N)

    static_rows, cat_rows = _sc_gather_all(
        static_table_0, static_table_1, static_table_2, static_table_3,
        stat_idx, cat_tab, cat_idx)
    g_mat = cat_rows.reshape(NC, T, B // 2, 2 * H)

    sout = pl.pallas_call(
        _static_body,
        grid=(4,),
        in_specs=[pl.BlockSpec((1, B, H), lambda j: (j, 0, 0))],
        out_specs=pl.BlockSpec((1, H, B), lambda j: (j, 0, 0)),
        out_shape=jax.ShapeDtypeStruct((4, H, B), f32),
    )(static_rows.reshape(4, B, H))
    static_out = sout.transpose(2, 0, 1)

    p_np = np.zeros((8 * H, NC * H), np.float32)
    for c in range(NC):
        for h in range(H):
            p_np[h * 8 + NR + c, c * H + h] = 1.0
    p_mat = jnp.asarray(p_np, dtype=jnp.bfloat16)

    eye84 = jnp.asarray(np.eye(8, 4, dtype=np.float32))
    kr_w = jnp.stack([kr_W_0[0], kr_W_1[0], kr_W_2[0], kr_W_3[0]], axis=1)
    wk_col = (kr_w[:, None, :] * eye84[None, :, :]).reshape(8 * H, NR)
    kb_col = jnp.stack(
        [kr_b_0, kr_b_1, kr_b_2, kr_b_3] + [jnp.zeros((H,), f32)] * 4,
        axis=-1).reshape(8 * H, 1)

    eye44 = jnp.asarray(np.eye(4, dtype=np.float32))
    obs_w = jnp.stack([obs_W_0[0], obs_W_1[0], obs_W_2[0], obs_W_3[0]], axis=1)
    wo_col = (obs_w[:, None, :] * eye44[None, :, :]).reshape(4 * H, NO)
    ob_col = jnp.stack([obs_b_0, obs_b_1, obs_b_2, obs_b_3],
                       axis=-1).reshape(4 * H, 1)

    xks = [x.T.reshape(T, 1, B) for x in kreals]
    xos = [x.T.reshape(T, 1, B) for x in obs]

    xspec = pl.BlockSpec((1, 1, B), lambda t: (t, 0, 0))
    known_p, obs_p = pl.pallas_call(
        _tc_body,
        grid=(T,),
        in_specs=[
            pl.BlockSpec((NC, 1, (B // 2) * (2 * H) // 128, 128),
                         lambda t: (0, t, 0, 0)),
            xspec, xspec, xspec, xspec, xspec, xspec, xspec, xspec,
            pl.BlockSpec((8 * H, NC * H), lambda t: (0, 0)),
            pl.BlockSpec((8 * H, NR), lambda t: (0, 0)),
            pl.BlockSpec((8 * H, 1), lambda t: (0, 0)),
            pl.BlockSpec((4 * H, NO), lambda t: (0, 0)),
            pl.BlockSpec((4 * H, 1), lambda t: (0, 0)),
        ],
        out_specs=[
            pl.BlockSpec((1, 8 * H, B), lambda t: (t, 0, 0)),
            pl.BlockSpec((1, 4 * H, B), lambda t: (t, 0, 0)),
        ],
        out_shape=[
            jax.ShapeDtypeStruct((T, 8 * H, B), f32),
            jax.ShapeDtypeStruct((T, 4 * H, B), f32),
        ],
    )(g_mat, *xks, *xos, p_mat, wk_col, kb_col, wo_col, ob_col)

    known = known_p.reshape(T, H, 8, B).transpose(3, 0, 1, 2)
    observed = obs_p.reshape(T, H, 4, B).transpose(3, 0, 1, 2)
    return (static_out, known, observed)


# speedup vs baseline: 3.7665x; 1.0386x over previous
import functools

import jax
import jax.numpy as jnp
import numpy as np
from jax.experimental import pallas as pl
from jax.experimental.pallas import tpu as pltpu
from jax.experimental.pallas import tpu_sc as plsc

B = 1024
T = 200
H = 64
N = B * T
NR = 4
NC = 4
NO = 4
CAT_VOCAB = 1000

GATHER_W = 128


_SC_MESH = plsc.VectorSubcoreMesh(core_axis_name="core", subcore_axis_name="subcore")
_SC_PARAMS = pltpu.CompilerParams(use_tc_tiling_on_sc=False)


def _sc_gather_static(st0, st1, st2, st3, stat_idx):
    @functools.partial(
        pl.kernel,
        out_type=jax.ShapeDtypeStruct((4 * B, H), jnp.float32),
        mesh=_SC_MESH,
        compiler_params=_SC_PARAMS,
    )
    def sc_kernel(t0_hbm, t1_hbm, t2_hbm, t3_hbm, stat_idx_hbm, stat_out_hbm):
        for j, tab in enumerate((t0_hbm, t1_hbm, t2_hbm, t3_hbm)):
            def stat_body(i_vmem, o_vmem, tab=tab):
                pltpu.sync_copy(tab.at[i_vmem.at[0]], o_vmem)

            pltpu.emit_pipeline(
                stat_body,
                grid=(B // GATHER_W,),
                in_specs=[pl.BlockSpec((1, GATHER_W), lambda i, j=j: (j, i))],
                out_specs=[pl.BlockSpec((GATHER_W, H),
                                        lambda i, j=j: (j * (B // GATHER_W) + i, 0))],
                core_axis_name=("core", "subcore"),
                dimension_semantics=(pltpu.PARALLEL,),
            )(stat_idx_hbm, stat_out_hbm)

    return sc_kernel(st0, st1, st2, st3, stat_idx)


def _sc_gather_cat(cat_tab, cat_idx):
    @functools.partial(
        pl.kernel,
        out_type=jax.ShapeDtypeStruct((NC * T * (B // 2), 2, H), jnp.float32),
        mesh=_SC_MESH,
        compiler_params=_SC_PARAMS,
    )
    def sc_kernel(cat_tab_hbm, cat_idx_hbm, cat_out_hbm):
        def cat_body(i_vmem, o_vmem):
            pltpu.sync_copy(cat_tab_hbm.at[i_vmem.at[0]], o_vmem.at[:, 0])

        pltpu.emit_pipeline(
            cat_body,
            grid=(NC * N // GATHER_W,),
            in_specs=[pl.BlockSpec((1, GATHER_W), lambda w: (0, w))],
            out_specs=[pl.BlockSpec(
                (GATHER_W, 1, H),
                lambda w: (w // 8 * (B // 2 // GATHER_W) + w % 4, w % 8 // 4, 0))],
            core_axis_name=("core", "subcore"),
            dimension_semantics=(pltpu.PARALLEL,),
        )(cat_idx_hbm, cat_out_hbm)

    return sc_kernel(cat_tab, cat_idx)


def _tc_body(g_ref, xk0, xk1, xk2, xk3, xo0, xo1, xo2, xo3, p_ref, wk_ref,
             kb_ref, wo_ref, ob_ref, known_ref, obs_ref):
    raw = g_ref[...].reshape(NC, B // 2, 2 * H)
    ga = raw[:, :, 0:H]
    gb = raw[:, :, H:2 * H]
    ta = jnp.transpose(ga, (0, 2, 1))
    tb = jnp.transpose(gb, (0, 2, 1))
    gt = jnp.concatenate([ta, tb], axis=-1).reshape(NC * H, B)
    hi = gt.astype(jnp.bfloat16)
    lo = (gt - hi.astype(jnp.float32)).astype(jnp.bfloat16)
    p = p_ref[...]
    acc = jnp.dot(p, hi, preferred_element_type=jnp.float32)
    acc = acc + jnp.dot(p, lo, preferred_element_type=jnp.float32)
    for c, xk in enumerate((xk0, xk1, xk2, xk3)):
        acc = acc + wk_ref[:, c:c + 1] * xk[0]
    known_ref[0] = acc + kb_ref[...]
    acc2 = wo_ref[:, 0:1] * xo0[0]
    for c, xo in enumerate((xo1, xo2, xo3)):
        acc2 = acc2 + wo_ref[:, c + 1:c + 2] * xo[0]
    obs_ref[0] = acc2 + ob_ref[...]


def _static_body(in_ref, out_ref):
    out_ref[0] = jnp.transpose(in_ref[0])


def kernel(static_0, static_1, static_2, static_3, known_real_0, known_real_1, known_real_2, known_real_3, known_cat_0, known_cat_1, known_cat_2, known_cat_3, observed_0, observed_1, observed_2, observed_3, static_table_0, static_table_1, static_table_2, static_table_3, known_cat_table_0, known_cat_table_1, known_cat_table_2, known_cat_table_3, kr_W_0, kr_b_0, kr_W_1, kr_b_1, kr_W_2, kr_b_2, kr_W_3, kr_b_3, obs_W_0, obs_b_0, obs_W_1, obs_b_1, obs_W_2, obs_b_2, obs_W_3, obs_b_3):
    f32 = jnp.float32
    statics = [static_0, static_1, static_2, static_3]
    kcats = [known_cat_0, known_cat_1, known_cat_2, known_cat_3]
    kreals = [known_real_0, known_real_1, known_real_2, known_real_3]
    obs = [observed_0, observed_1, observed_2, observed_3]

    stat_idx = jnp.stack([s.astype(jnp.int32) for s in statics], axis=0)
    cat_tab = jnp.concatenate(
        [known_cat_table_0, known_cat_table_1, known_cat_table_2,
         known_cat_table_3], axis=0)
    cat_idx = jnp.concatenate(
        [k.astype(jnp.int32).T.reshape(N) + CAT_VOCAB * c
         for c, k in enumerate(kcats)]).reshape(1, NC * N)

    cat_rows = _sc_gather_cat(cat_tab, cat_idx)
    static_rows = _sc_gather_static(
        static_table_0, static_table_1, static_table_2, static_table_3, stat_idx)
    g_mat = cat_rows.reshape(NC, T, B // 2, 2 * H)

    sout = pl.pallas_call(
        _static_body,
        grid=(4,),
        in_specs=[pl.BlockSpec((1, B, H), lambda j: (j, 0, 0))],
        out_specs=pl.BlockSpec((1, H, B), lambda j: (j, 0, 0)),
        out_shape=jax.ShapeDtypeStruct((4, H, B), f32),
    )(static_rows.reshape(4, B, H))
    static_out = sout.transpose(2, 0, 1)

    p_np = np.zeros((8 * H, NC * H), np.float32)
    for c in range(NC):
        for h in range(H):
            p_np[h * 8 + NR + c, c * H + h] = 1.0
    p_mat = jnp.asarray(p_np, dtype=jnp.bfloat16)

    eye84 = jnp.asarray(np.eye(8, 4, dtype=np.float32))
    kr_w = jnp.stack([kr_W_0[0], kr_W_1[0], kr_W_2[0], kr_W_3[0]], axis=1)
    wk_col = (kr_w[:, None, :] * eye84[None, :, :]).reshape(8 * H, NR)
    kb_col = jnp.stack(
        [kr_b_0, kr_b_1, kr_b_2, kr_b_3] + [jnp.zeros((H,), f32)] * 4,
        axis=-1).reshape(8 * H, 1)

    eye44 = jnp.asarray(np.eye(4, dtype=np.float32))
    obs_w = jnp.stack([obs_W_0[0], obs_W_1[0], obs_W_2[0], obs_W_3[0]], axis=1)
    wo_col = (obs_w[:, None, :] * eye44[None, :, :]).reshape(4 * H, NO)
    ob_col = jnp.stack([obs_b_0, obs_b_1, obs_b_2, obs_b_3],
                       axis=-1).reshape(4 * H, 1)

    xks = [x.T.reshape(T, 1, B) for x in kreals]
    xos = [x.T.reshape(T, 1, B) for x in obs]

    xspec = pl.BlockSpec((1, 1, B), lambda t: (t, 0, 0))
    known_p, obs_p = pl.pallas_call(
        _tc_body,
        grid=(T,),
        in_specs=[
            pl.BlockSpec((NC, 1, (B // 2) * (2 * H) // 128, 128),
                         lambda t: (0, t, 0, 0)),
            xspec, xspec, xspec, xspec, xspec, xspec, xspec, xspec,
            pl.BlockSpec((8 * H, NC * H), lambda t: (0, 0)),
            pl.BlockSpec((8 * H, NR), lambda t: (0, 0)),
            pl.BlockSpec((8 * H, 1), lambda t: (0, 0)),
            pl.BlockSpec((4 * H, NO), lambda t: (0, 0)),
            pl.BlockSpec((4 * H, 1), lambda t: (0, 0)),
        ],
        out_specs=[
            pl.BlockSpec((1, 8 * H, B), lambda t: (t, 0, 0)),
            pl.BlockSpec((1, 4 * H, B), lambda t: (t, 0, 0)),
        ],
        out_shape=[
            jax.ShapeDtypeStruct((T, 8 * H, B), f32),
            jax.ShapeDtypeStruct((T, 4 * H, B), f32),
        ],
    )(g_mat, *xks, *xos, p_mat, wk_col, kb_col, wo_col, ob_col)

    known = known_p.reshape(T, H, 8, B).transpose(3, 0, 1, 2)
    observed = obs_p.reshape(T, H, 4, B).transpose(3, 0, 1, 2)
    return (static_out, known, observed)


# speedup vs baseline: 4.3687x; 1.1599x over previous
import functools

import jax
import jax.numpy as jnp
import numpy as np
from jax.experimental import pallas as pl
from jax.experimental.pallas import tpu as pltpu
from jax.experimental.pallas import tpu_sc as plsc

B = 1024
T = 200
H = 64
N = B * T
NR = 4
NC = 4
NO = 4
CAT_VOCAB = 1000

GATHER_W = 128


_SC_PARAMS = pltpu.CompilerParams(use_tc_tiling_on_sc=False)


def _sc_mesh():
    return plsc.VectorSubcoreMesh(core_axis_name="core", subcore_axis_name="subcore")


def _sc_gather_static(st0, st1, st2, st3, stat_idx):
    @functools.partial(
        pl.kernel,
        out_type=jax.ShapeDtypeStruct((4 * B, H), jnp.float32),
        mesh=_sc_mesh(),
        compiler_params=_SC_PARAMS,
    )
    def sc_kernel(t0_hbm, t1_hbm, t2_hbm, t3_hbm, stat_idx_hbm, stat_out_hbm):
        for j, tab in enumerate((t0_hbm, t1_hbm, t2_hbm, t3_hbm)):
            def stat_body(i_vmem, o_vmem, tab=tab):
                pltpu.sync_copy(tab.at[i_vmem.at[0]], o_vmem)

            pltpu.emit_pipeline(
                stat_body,
                grid=(B // GATHER_W,),
                in_specs=[pl.BlockSpec((1, GATHER_W), lambda i, j=j: (j, i))],
                out_specs=[pl.BlockSpec((GATHER_W, H),
                                        lambda i, j=j: (j * (B // GATHER_W) + i, 0))],
                core_axis_name=("core", "subcore"),
                dimension_semantics=(pltpu.PARALLEL,),
            )(stat_idx_hbm, stat_out_hbm)

    return sc_kernel(st0, st1, st2, st3, stat_idx)


def _sc_gather_cat(cat_tab, cat_idx, k, tk):
    wpt = B // GATHER_W
    wpc = tk * wpt

    @functools.partial(
        pl.kernel,
        out_type=jax.ShapeDtypeStruct((NC * tk * (B // 2), 2, H), jnp.float32),
        mesh=_sc_mesh(),
        compiler_params=_SC_PARAMS,
    )
    def sc_kernel(cat_tab_hbm, cat_idx_hbm, cat_out_hbm):
        def cat_body(i_vmem, o_vmem):
            pltpu.sync_copy(cat_tab_hbm.at[i_vmem.at[0]], o_vmem.at[:, 0])

        pltpu.emit_pipeline(
            cat_body,
            grid=(NC * tk * B // GATHER_W,),
            in_specs=[pl.BlockSpec(
                (1, GATHER_W),
                lambda w: (0, w // wpc * (T * wpt) + k * tk * wpt + w % wpc))],
            out_specs=[pl.BlockSpec(
                (GATHER_W, 1, H),
                lambda w: (w // 8 * (B // 2 // GATHER_W) + w % 4, w % 8 // 4, 0))],
            core_axis_name=("core", "subcore"),
            dimension_semantics=(pltpu.PARALLEL,),
        )(cat_idx_hbm, cat_out_hbm)

    return sc_kernel(cat_tab, cat_idx)


def _tc_body(g_ref, xk0, xk1, xk2, xk3, xo0, xo1, xo2, xo3, p_ref, wk_ref,
             kb_ref, wo_ref, ob_ref, known_ref, obs_ref):
    raw = g_ref[...].reshape(NC, B // 2, 2 * H)
    ga = raw[:, :, 0:H]
    gb = raw[:, :, H:2 * H]
    ta = jnp.transpose(ga, (0, 2, 1))
    tb = jnp.transpose(gb, (0, 2, 1))
    gt = jnp.concatenate([ta, tb], axis=-1).reshape(NC * H, B)
    hi = gt.astype(jnp.bfloat16)
    lo = (gt - hi.astype(jnp.float32)).astype(jnp.bfloat16)
    p = p_ref[...]
    acc = jnp.dot(p, hi, preferred_element_type=jnp.float32)
    acc = acc + jnp.dot(p, lo, preferred_element_type=jnp.float32)
    for c, xk in enumerate((xk0, xk1, xk2, xk3)):
        acc = acc + wk_ref[:, c:c + 1] * xk[0]
    known_ref[0] = acc + kb_ref[...]
    acc2 = wo_ref[:, 0:1] * xo0[0]
    for c, xo in enumerate((xo1, xo2, xo3)):
        acc2 = acc2 + wo_ref[:, c + 1:c + 2] * xo[0]
    obs_ref[0] = acc2 + ob_ref[...]


def _tc_body_alias(g_ref, xk0, xk1, xk2, xk3, xo0, xo1, xo2, xo3, p_ref,
                   wk_ref, kb_ref, wo_ref, ob_ref, ka_ref, oa_ref,
                   known_ref, obs_ref):
    del ka_ref, oa_ref
    _tc_body(g_ref, xk0, xk1, xk2, xk3, xo0, xo1, xo2, xo3, p_ref,
             wk_ref, kb_ref, wo_ref, ob_ref, known_ref, obs_ref)


def _static_body(in_ref, out_ref):
    out_ref[0] = jnp.transpose(in_ref[0])


def kernel(static_0, static_1, static_2, static_3, known_real_0, known_real_1, known_real_2, known_real_3, known_cat_0, known_cat_1, known_cat_2, known_cat_3, observed_0, observed_1, observed_2, observed_3, static_table_0, static_table_1, static_table_2, static_table_3, known_cat_table_0, known_cat_table_1, known_cat_table_2, known_cat_table_3, kr_W_0, kr_b_0, kr_W_1, kr_b_1, kr_W_2, kr_b_2, kr_W_3, kr_b_3, obs_W_0, obs_b_0, obs_W_1, obs_b_1, obs_W_2, obs_b_2, obs_W_3, obs_b_3):
    f32 = jnp.float32
    statics = [static_0, static_1, static_2, static_3]
    kcats = [known_cat_0, known_cat_1, known_cat_2, known_cat_3]
    kreals = [known_real_0, known_real_1, known_real_2, known_real_3]
    obs = [observed_0, observed_1, observed_2, observed_3]

    stat_idx = jnp.stack([s.astype(jnp.int32) for s in statics], axis=0)
    cat_tab = jnp.concatenate(
        [known_cat_table_0, known_cat_table_1, known_cat_table_2,
         known_cat_table_3], axis=0)
    cat_idx = jnp.concatenate(
        [k.astype(jnp.int32).T.reshape(N) + CAT_VOCAB * c
         for c, k in enumerate(kcats)]).reshape(1, NC * N)

    static_rows = _sc_gather_static(
        static_table_0, static_table_1, static_table_2, static_table_3, stat_idx)

    sout = pl.pallas_call(
        _static_body,
        grid=(4,),
        in_specs=[pl.BlockSpec((1, B, H), lambda j: (j, 0, 0))],
        out_specs=pl.BlockSpec((1, H, B), lambda j: (j, 0, 0)),
        out_shape=jax.ShapeDtypeStruct((4, H, B), f32),
    )(static_rows.reshape(4, B, H))
    static_out = sout.transpose(2, 0, 1)

    p_np = np.zeros((8 * H, NC * H), np.float32)
    for c in range(NC):
        for h in range(H):
            p_np[h * 8 + NR + c, c * H + h] = 1.0
    p_mat = jnp.asarray(p_np, dtype=jnp.bfloat16)

    eye84 = jnp.asarray(np.eye(8, 4, dtype=np.float32))
    kr_w = jnp.stack([kr_W_0[0], kr_W_1[0], kr_W_2[0], kr_W_3[0]], axis=1)
    wk_col = (kr_w[:, None, :] * eye84[None, :, :]).reshape(8 * H, NR)
    kb_col = jnp.stack(
        [kr_b_0, kr_b_1, kr_b_2, kr_b_3] + [jnp.zeros((H,), f32)] * 4,
        axis=-1).reshape(8 * H, 1)

    eye44 = jnp.asarray(np.eye(4, dtype=np.float32))
    obs_w = jnp.stack([obs_W_0[0], obs_W_1[0], obs_W_2[0], obs_W_3[0]], axis=1)
    wo_col = (obs_w[:, None, :] * eye44[None, :, :]).reshape(4 * H, NO)
    ob_col = jnp.stack([obs_b_0, obs_b_1, obs_b_2, obs_b_3],
                       axis=-1).reshape(4 * H, 1)

    xks = [x.T.reshape(T, 1, B) for x in kreals]
    xos = [x.T.reshape(T, 1, B) for x in obs]

    kch = 4
    tk = T // kch
    out_shape = [
        jax.ShapeDtypeStruct((T, 8 * H, B), f32),
        jax.ShapeDtypeStruct((T, 4 * H, B), f32),
    ]
    known_p = obs_p = None
    for k in range(kch):
        cat_rows_k = _sc_gather_cat(cat_tab, cat_idx, k, tk)
        g_mat_k = cat_rows_k.reshape(NC, tk, B // 2, 2 * H)
        xspec = pl.BlockSpec((1, 1, B), lambda t, k=k: (k * tk + t, 0, 0))
        in_specs = [
            pl.BlockSpec((NC, 1, B // 2, 2 * H), lambda t: (0, t, 0, 0)),
            xspec, xspec, xspec, xspec, xspec, xspec, xspec, xspec,
            pl.BlockSpec((8 * H, NC * H), lambda t: (0, 0)),
            pl.BlockSpec((8 * H, NR), lambda t: (0, 0)),
            pl.BlockSpec((8 * H, 1), lambda t: (0, 0)),
            pl.BlockSpec((4 * H, NO), lambda t: (0, 0)),
            pl.BlockSpec((4 * H, 1), lambda t: (0, 0)),
        ]
        out_specs = [
            pl.BlockSpec((1, 8 * H, B), lambda t, k=k: (k * tk + t, 0, 0)),
            pl.BlockSpec((1, 4 * H, B), lambda t, k=k: (k * tk + t, 0, 0)),
        ]
        ins = [g_mat_k, *xks, *xos, p_mat, wk_col, kb_col, wo_col, ob_col]
        if k == 0:
            known_p, obs_p = pl.pallas_call(
                _tc_body, grid=(tk,), in_specs=in_specs,
                out_specs=out_specs, out_shape=out_shape,
            )(*ins)
        else:
            anyspec = pl.BlockSpec(memory_space=pl.ANY)
            known_p, obs_p = pl.pallas_call(
                _tc_body_alias, grid=(tk,),
                in_specs=in_specs + [anyspec, anyspec],
                out_specs=out_specs, out_shape=out_shape,
                input_output_aliases={len(ins): 0, len(ins) + 1: 1},
            )(*ins, known_p, obs_p)

    known = known_p.reshape(T, H, 8, B).transpose(3, 0, 1, 2)
    observed = obs_p.reshape(T, H, 4, B).transpose(3, 0, 1, 2)
    return (static_out, known, observed)
